# baseline (speedup 1.0000x reference)
import jax
import jax.numpy as jnp
from jax.experimental import pallas as pl
from jax.experimental.pallas import tpu as pltpu

_BK = 4096
_NBUF = 4


def _pool_kernel(p_ref, x_hbm, o_ref, acc_ref, xbufs, sems):
    c = pl.program_id(0)
    nsteps = pl.num_programs(0)
    nb = acc_ref.shape[0]
    s = nb // 2

    h = _BK // 2

    def dma(block, buf, half):
        return pltpu.make_async_copy(
            x_hbm.at[pl.ds(block * _BK + half * h, h)],
            xbufs.at[buf, pl.ds(half * h, h)], sems.at[buf, half])

    def start(block, buf):
        dma(block, buf, 0).start()
        dma(block, buf, 1).start()

    def wait(block, buf):
        dma(block, buf, 0).wait()
        dma(block, buf, 1).wait()

    @pl.when(c == 0)
    def _():
        acc_ref[...] = jnp.zeros_like(acc_ref)
        for k in range(_NBUF):
            start(k, k)

    p = p_ref[...]
    for k in range(_NBUF):
        @pl.when(c % _NBUF == k)
        def _(k=k):
            wait(c, k)
            row_ids = (jax.lax.broadcasted_iota(jnp.int32, (nb, _BK), 1)
                       + c * _BK)
            mask = (row_ids < p).astype(jnp.float32)
            acc_ref[...] += jax.lax.dot_general(
                mask, xbufs[k], (((1,), (0,)), ((), ())),
                preferred_element_type=jnp.float32)

            @pl.when(c + _NBUF < nsteps)
            def _():
                start(c + _NBUF, k)

    @pl.when(c == nsteps - 1)
    def _():
        acc = acc_ref[...]
        cnt = (p[s:] - p[:s]).astype(jnp.float32)
        o_ref[...] = (acc[s:, :] - acc[:s, :]) / cnt + jnp.float32(0.006)


def kernel(input, graph):
    n, d = input.shape
    s = graph.shape[0]
    g = graph.astype(jnp.int32)
    p = jnp.concatenate([g[:, 0], g[:, 1] + 1]).reshape(2 * s, 1)
    return pl.pallas_call(
        _pool_kernel,
        grid=(n // _BK,),
        in_specs=[
            pl.BlockSpec((2 * s, 1), lambda c: (0, 0)),
            pl.BlockSpec(memory_space=pl.ANY),
        ],
        out_specs=pl.BlockSpec((s, d), lambda c: (0, 0)),
        out_shape=jax.ShapeDtypeStruct((s, d), jnp.float32),
        scratch_shapes=[
            pltpu.VMEM((2 * s, d), jnp.float32),
            pltpu.VMEM((_NBUF, _BK, d), jnp.float32),
            pltpu.SemaphoreType.DMA((_NBUF, 2)),
        ],
    )(p, input)


# speedup vs baseline: 1.0746x; 1.0746x over previous
import jax
import jax.numpy as jnp
from jax.experimental import pallas as pl
from jax.experimental.pallas import tpu as pltpu

_BK = 8192


def _pool_kernel(p_ref, x_ref, o_ref, acc_ref):
    c = pl.program_id(0)
    nc = pl.num_programs(0)
    nb = acc_ref.shape[0]
    s = nb // 2

    @pl.when(c == 0)
    def _():
        acc_ref[...] = jnp.zeros_like(acc_ref)

    p = p_ref[...]
    row_ids = jax.lax.broadcasted_iota(jnp.int32, (nb, _BK), 1) + c * _BK
    mask = (row_ids < p).astype(jnp.float32)
    acc_ref[...] += jax.lax.dot_general(
        mask, x_ref[...], (((1,), (0,)), ((), ())),
        preferred_element_type=jnp.float32)

    @pl.when(c == nc - 1)
    def _():
        acc = acc_ref[...]
        cnt = (p[s:] - p[:s]).astype(jnp.float32)
        o_ref[...] = (acc[s:, :] - acc[:s, :]) / cnt + jnp.float32(0.006)


def kernel(input, graph):
    n, d = input.shape
    s = graph.shape[0]
    g = graph.astype(jnp.int32)
    p = jnp.concatenate([g[:, 0], g[:, 1] + 1]).reshape(2 * s, 1)
    return pl.pallas_call(
        _pool_kernel,
        grid=(n // _BK,),
        in_specs=[
            pl.BlockSpec((2 * s, 1), lambda c: (0, 0)),
            pl.BlockSpec((_BK, d), lambda c: (c, 0)),
        ],
        out_specs=pl.BlockSpec((s, d), lambda c: (0, 0)),
        out_shape=jax.ShapeDtypeStruct((s, d), jnp.float32),
        scratch_shapes=[pltpu.VMEM((2 * s, d), jnp.float32)],
    )(p, input)
